# baseline (speedup 1.0000x reference)
import jax
import jax.numpy as jnp
from jax import lax
from jax.experimental import pallas as pl
from jax.experimental.pallas import tpu as pltpu
from jax.experimental.pallas import tpu_sc as plsc

B = 4096
T = 200
D = 64
N = B * T
WEEK = 7
DAY = 288

NUM_CORES = 2
NUM_SUBCORES = 16
NW = NUM_CORES * NUM_SUBCORES
ROWS_W = B // NW
R = 2
C = R * T
CHUNKS = ROWS_W // R
NGROUPS = 13


def _body(pw_hbm, pd_hbm, dow_hbm, tod_hbm, out_hbm,
          dow_l, tod_l, iw0, iw1, id0, id1, ob0, ob1,
          si0, si1, sob0, sob1, stab):
  iw = (iw0, iw1)
  idd = (id0, id1)
  ob = (ob0, ob1)
  si = (si0, si1)
  sob = (sob0, sob1)

  cid = lax.axis_index("c")
  sid_ = lax.axis_index("s")
  wid = sid_ * NUM_CORES + cid
  row0 = wid * ROWS_W

  cp1 = pltpu.async_copy(dow_hbm, dow_l, stab)
  cp2 = pltpu.async_copy(tod_hbm, tod_l, stab)

  def idx_start(i, s):
    base = (row0 + i * R) * T
    pltpu.async_copy(pw_hbm.at[pl.ds(base, C)], iw[s], si[s])
    pltpu.async_copy(pd_hbm.at[pl.ds(base, C)], idd[s], si[s])

  def idx_wait(s):
    pltpu.make_async_copy(pw_hbm.at[pl.ds(0, C)], iw[s], si[s]).wait()
    pltpu.make_async_copy(pd_hbm.at[pl.ds(0, C)], idd[s], si[s]).wait()

  def out_start(i, s):
    pltpu.async_copy(ob[s], out_hbm.at[pl.ds(row0 + i * R, R)], sob[s])

  def out_wait(s):
    pltpu.make_async_copy(ob[s], out_hbm.at[pl.ds(0, R)], sob[s]).wait()

  def compute(s):
    for rr in range(R):
      @plsc.parallel_loop(0, NGROUPS, 1)
      def _(g):
        t0 = jnp.minimum(g * 16, T - 16)
        wv = iw[s][pl.ds(rr * T + t0, 16)] * D
        dv = idd[s][pl.ds(rr * T + t0, 16)] * D
        for jj in range(16):
          w = wv[jj]
          d = dv[jj]
          t = t0 + jj
          for j in range(D // 16):
            sl = pl.ds(j * 16, 16)
            ob[s][rr, t, 0, sl] = (dow_l[pl.ds(w + j * 16, 16)]
                                   + tod_l[pl.ds(d + j * 16, 16)])

  idx_start(0, 0)
  idx_start(1, 1)
  cp1.wait()
  cp2.wait()

  def step(i, s):
    idx_wait(s)

    @pl.when(i >= 2)
    def _():
      out_wait(s)

    compute(s)
    out_start(i, s)

    @pl.when(i + 2 < CHUNKS)
    def _():
      idx_start(i + 2, s)

  def group(g, carry):
    step(2 * g, 0)
    step(2 * g + 1, 1)
    return carry

  lax.fori_loop(0, CHUNKS // 2, group, None)
  out_wait(0)
  out_wait(1)


@jax.jit
def _run(pw, pd, dow_table, tod_table):
  mesh = plsc.VectorSubcoreMesh(core_axis_name="c", subcore_axis_name="s")
  k = pl.kernel(
      _body,
      out_type=jax.ShapeDtypeStruct((B, T, 1, D), jnp.float32),
      mesh=mesh,
      scratch_types=[
          pltpu.VMEM((WEEK * D,), jnp.float32),
          pltpu.VMEM((DAY * D,), jnp.float32),
          pltpu.VMEM((C,), jnp.int32),
          pltpu.VMEM((C,), jnp.int32),
          pltpu.VMEM((C,), jnp.int32),
          pltpu.VMEM((C,), jnp.int32),
          pltpu.VMEM((R, T, 1, D), jnp.float32),
          pltpu.VMEM((R, T, 1, D), jnp.float32),
          pltpu.SemaphoreType.DMA,
          pltpu.SemaphoreType.DMA,
          pltpu.SemaphoreType.DMA,
          pltpu.SemaphoreType.DMA,
          pltpu.SemaphoreType.DMA,
      ],
      compiler_params=pltpu.CompilerParams(use_tc_tiling_on_sc=False),
  )
  return k(pw, pd, dow_table, tod_table)


def kernel(pos_w, pos_d, dow_table, tod_table):
  pw = pos_w.reshape(N).astype(jnp.int32)
  pd = pos_d.reshape(N).astype(jnp.int32)
  return _run(pw, pd, dow_table.reshape(WEEK * D), tod_table.reshape(DAY * D))


# speedup vs baseline: 3.0695x; 3.0695x over previous
import jax
import jax.numpy as jnp
from jax import lax
from jax.experimental import pallas as pl
from jax.experimental.pallas import tpu as pltpu
from jax.experimental.pallas import tpu_sc as plsc

B = 4096
T = 200
D = 64
N = B * T
WEEK = 7
DAY = 288

NUM_CORES = 2
NUM_SUBCORES = 16
NW = NUM_CORES * NUM_SUBCORES
PER_W = N // NW
C = 256
CHUNKS = PER_W // C


def _body(pw_hbm, pd_hbm, dow_hbm, tod_hbm, out_hbm,
          dow_l, tod_l, iw0, iw1, id0, id1, ob0, ob1,
          si0, si1, sob0, sob1, stab):
  iw = (iw0, iw1)
  idd = (id0, id1)
  ob = (ob0, ob1)
  si = (si0, si1)
  sob = (sob0, sob1)

  cid = lax.axis_index("c")
  sid_ = lax.axis_index("s")
  wid = sid_ * NUM_CORES + cid
  base0 = wid * PER_W

  cp1 = pltpu.async_copy(dow_hbm, dow_l, stab)
  cp2 = pltpu.async_copy(tod_hbm, tod_l, stab)

  def idx_start(i, s):
    base = base0 + i * C
    pltpu.async_copy(pw_hbm.at[pl.ds(base, C)], iw[s], si[s])
    pltpu.async_copy(pd_hbm.at[pl.ds(base, C)], idd[s], si[s])

  def idx_wait(s):
    pltpu.make_async_copy(pw_hbm.at[pl.ds(0, C)], iw[s], si[s]).wait()
    pltpu.make_async_copy(pd_hbm.at[pl.ds(0, C)], idd[s], si[s]).wait()

  def out_start(i, s):
    base = base0 + i * C
    pltpu.async_copy(ob[s], out_hbm.at[pl.ds(base, C)], sob[s])

  def out_wait(s):
    pltpu.make_async_copy(ob[s], out_hbm.at[pl.ds(0, C)], sob[s]).wait()

  def compute(s):
    @plsc.parallel_loop(0, C // 16, 1, unroll=2)
    def _(g):
      wv = iw[s][pl.ds(g * 16, 16)] * D
      dv = idd[s][pl.ds(g * 16, 16)] * D
      for jj in range(16):
        w = wv[jj]
        d = dv[jj]
        r = g * 16 + jj
        for j in range(D // 16):
          sl = pl.ds(j * 16, 16)
          ob[s][r, sl] = dow_l[pl.ds(w + j * 16, 16)] + tod_l[pl.ds(d + j * 16, 16)]

  idx_start(0, 0)
  idx_start(1, 1)
  cp1.wait()
  cp2.wait()

  def step(i, s):
    idx_wait(s)

    @pl.when(i >= 2)
    def _():
      out_wait(s)

    compute(s)
    out_start(i, s)

    @pl.when(i + 2 < CHUNKS)
    def _():
      idx_start(i + 2, s)

  def group(g, carry):
    step(2 * g, 0)
    step(2 * g + 1, 1)
    return carry

  lax.fori_loop(0, CHUNKS // 2, group, None)
  out_wait(0)
  out_wait(1)


@jax.jit
def _run(pw, pd, dow_table, tod_table):
  mesh = plsc.VectorSubcoreMesh(core_axis_name="c", subcore_axis_name="s")
  k = pl.kernel(
      _body,
      out_type=jax.ShapeDtypeStruct((N, D), jnp.float32),
      mesh=mesh,
      scratch_types=[
          pltpu.VMEM((WEEK * D,), jnp.float32),
          pltpu.VMEM((DAY * D,), jnp.float32),
          pltpu.VMEM((C,), jnp.int32),
          pltpu.VMEM((C,), jnp.int32),
          pltpu.VMEM((C,), jnp.int32),
          pltpu.VMEM((C,), jnp.int32),
          pltpu.VMEM((C, D), jnp.float32),
          pltpu.VMEM((C, D), jnp.float32),
          pltpu.SemaphoreType.DMA,
          pltpu.SemaphoreType.DMA,
          pltpu.SemaphoreType.DMA,
          pltpu.SemaphoreType.DMA,
          pltpu.SemaphoreType.DMA,
      ],
      compiler_params=pltpu.CompilerParams(use_tc_tiling_on_sc=True),
  )
  return k(pw, pd, dow_table, tod_table)


def kernel(pos_w, pos_d, dow_table, tod_table):
  pw = pos_w.reshape(N).astype(jnp.int32)
  pd = pos_d.reshape(N).astype(jnp.int32)
  out = _run(pw, pd, dow_table.reshape(WEEK * D), tod_table.reshape(DAY * D))
  return out.reshape(B, T, 1, D)
